# Initial kernel scaffold; baseline (speedup 1.0000x reference)
import jax
import jax.numpy as jnp
from jax.experimental import pallas as pl


def kernel(seq1, adj_index, adj_weight, adj_ori_index, adj_ori_weight, neg_num, neg_samples, samp_bias1, samp_bias2, W1, b1, a1, W2, b2, a2, a3, Wd1, bd1, Wd2, bd2):
    raise NotImplementedError("write your pallas kernel here")


# speedup vs baseline: 3.1047x; 3.1047x over previous
import functools

import jax
import jax.numpy as jnp
from jax import lax
from jax.experimental import pallas as pl
from jax.experimental.pallas import tpu as pltpu
from jax.experimental.pallas import tpu_sc as plsc

N = 10000
E = 320000
D = 128
NEG = 5
NC, NS = 2, 16
NW = NC * NS
EC = 80
ECPW = E // (NW * EC)
ROWS_PT = 624
ROWS_TAIL = N - NS * ROWS_PT

GC = 112
GCPW = 14
NEG_PAD = NW * GCPW * GC


def _sc_mesh():
    return plsc.VectorSubcoreMesh(core_axis_name="c", subcore_axis_name="s")


def _spmm_partials(sd3, w3, feats, zeros):
    @functools.partial(
        pl.kernel,
        mesh=_sc_mesh(),
        out_type=jax.ShapeDtypeStruct((NC, N, D), jnp.float32),
        scratch_types=[
            pltpu.VMEM((2, EC), jnp.int32),
            pltpu.VMEM((ECPW * EC,), jnp.float32),
            pltpu.VMEM((EC, D), jnp.float32),
            pltpu.VMEM_SHARED((N, D), jnp.float32),
            pltpu.SemaphoreType.DMA,
        ],
    )
    def k(sd_h, w_h, x_h, z_h, out_h, sd_v, w_v, rows_v, acc_s, sem):
        c = lax.axis_index("c")
        s = lax.axis_index("s")
        wid = s * NC + c
        pltpu.sync_copy(z_h.at[pl.ds(s * ROWS_PT, ROWS_PT)],
                        acc_s.at[pl.ds(s * ROWS_PT, ROWS_PT)])

        @pl.when(s == NS - 1)
        def _zero_tail():
            pltpu.sync_copy(z_h.at[pl.ds(NS * ROWS_PT, ROWS_TAIL)],
                            acc_s.at[pl.ds(NS * ROWS_PT, ROWS_TAIL)])
        pltpu.sync_copy(w_h.at[wid], w_v)
        plsc.subcore_barrier()

        dnums = lax.GatherDimensionNumbers(
            offset_dims=(), collapsed_slice_dims=(0,), start_index_map=(0,))

        def chunk_body(t, carry):
            pltpu.sync_copy(sd_h.at[wid, t], sd_v)
            pltpu.async_copy(x_h.at[sd_v.at[0]], rows_v, sem).wait()

            def grp_body(gi, carry2):
                wv = w_v[pl.ds(t * EC + gi * 16, 16)]
                for j in range(16):
                    wb = lax.gather(
                        wv, jnp.full((16, 1), j, jnp.int32), dnums, (1,),
                        mode=lax.GatherScatterMode.PROMISE_IN_BOUNDS)
                    e = gi * 16 + j
                    for g in range(D // 16):
                        rows_v[e, pl.ds(g * 16, 16)] = (
                            rows_v[e, pl.ds(g * 16, 16)] * wb)
                return carry2

            lax.fori_loop(0, EC // 16, grp_body, 0)
            pltpu.sync_copy(rows_v, acc_s.at[sd_v.at[1]], add=True)
            return carry

        lax.fori_loop(0, ECPW, chunk_body, 0)
        plsc.subcore_barrier()
        pltpu.sync_copy(acc_s.at[pl.ds(s * ROWS_PT, ROWS_PT)],
                        out_h.at[c, pl.ds(s * ROWS_PT, ROWS_PT)])

        @pl.when(s == NS - 1)
        def _write_tail():
            pltpu.sync_copy(acc_s.at[pl.ds(NS * ROWS_PT, ROWS_TAIL)],
                            out_h.at[c, pl.ds(NS * ROWS_PT, ROWS_TAIL)])

    return k(sd3, w3, feats, zeros)


def _gather_rows2(idx3, tab_a, tab_b):
    @functools.partial(
        pl.kernel,
        mesh=_sc_mesh(),
        out_type=(jax.ShapeDtypeStruct((NEG_PAD, D), jnp.float32),
                  jax.ShapeDtypeStruct((NEG_PAD, D), jnp.float32)),
        scratch_types=[
            pltpu.VMEM((GCPW, GC), jnp.int32),
            pltpu.VMEM((GC, D), jnp.float32),
            pltpu.VMEM((GC, D), jnp.float32),
            pltpu.SemaphoreType.DMA,
            pltpu.SemaphoreType.DMA,
        ],
    )
    def k(idx_h, a_h, b_h, oa_h, ob_h, idx_v, buf_a, buf_b, sem_a, sem_b):
        c = lax.axis_index("c")
        s = lax.axis_index("s")
        wid = s * NC + c
        base = wid * (GCPW * GC)
        pltpu.sync_copy(idx_h.at[wid], idx_v)

        def body(t, carry):
            cpa = pltpu.async_copy(a_h.at[idx_v.at[t]], buf_a, sem_a)
            cpb = pltpu.async_copy(b_h.at[idx_v.at[t]], buf_b, sem_b)
            cpa.wait()
            pltpu.sync_copy(buf_a, oa_h.at[pl.ds(base + t * GC, GC)])
            cpb.wait()
            pltpu.sync_copy(buf_b, ob_h.at[pl.ds(base + t * GC, GC)])
            return carry

        lax.fori_loop(0, GCPW, body, 0)

    return k(idx3, tab_a, tab_b)


RB = 1000
NRB = N // RB


def _proj2(x, W1, Wd1):
    def body(x_r, w1_r, wd1_r, o1_r, o2_r):
        xb = x_r[...]
        o1_r[...] = jnp.dot(xb, w1_r[...], preferred_element_type=jnp.float32)
        o2_r[...] = jnp.dot(xb, wd1_r[...], preferred_element_type=jnp.float32)

    return pl.pallas_call(
        body,
        grid=(NRB,),
        in_specs=[
            pl.BlockSpec((RB, D), lambda i: (i, 0)),
            pl.BlockSpec((D, D), lambda i: (0, 0)),
            pl.BlockSpec((D, D), lambda i: (0, 0)),
        ],
        out_specs=[
            pl.BlockSpec((RB, D), lambda i: (i, 0)),
            pl.BlockSpec((RB, D), lambda i: (i, 0)),
        ],
        out_shape=[
            jax.ShapeDtypeStruct((N, D), jnp.float32),
            jax.ShapeDtypeStruct((N, D), jnp.float32),
        ],
    )(x, W1, Wd1)


def _gcn1_tail(p1, b1, a1, W2):
    def body(pa_r, pb_r, b_r, a_r, w2_r, o_r):
        v = pa_r[...] + pb_r[...] + b_r[...]
        h = jnp.where(v >= 0, v, a_r[0, 0] * v)
        o_r[...] = jnp.dot(h, w2_r[...], preferred_element_type=jnp.float32)

    return pl.pallas_call(
        body,
        grid=(NRB,),
        in_specs=[
            pl.BlockSpec((RB, D), lambda i: (i, 0)),
            pl.BlockSpec((RB, D), lambda i: (i, 0)),
            pl.BlockSpec((1, D), lambda i: (0, 0)),
            pl.BlockSpec((1, 1), lambda i: (0, 0), memory_space=pltpu.SMEM),
            pl.BlockSpec((D, D), lambda i: (0, 0)),
        ],
        out_specs=pl.BlockSpec((RB, D), lambda i: (i, 0)),
        out_shape=jax.ShapeDtypeStruct((N, D), jnp.float32),
    )(p1[0], p1[1], b1, a1, W2)


def _stage2(p2, b2, a2, p3, a3, Wd2, u1, samp_bias1, bd1, bd2):
    def body(p2a_r, p2b_r, b2_r, a2_r, p3a_r, p3b_r, a3_r, wd2_r, u1_r,
             sb1_r, bd1_r, bd2_r, h2_r, t2_r, mp_r, lp_r):
        v2 = p2a_r[...] + p2b_r[...] + b2_r[...]
        h2 = jnp.where(v2 >= 0, v2, a2_r[0, 0] * v2)
        h2_r[...] = h2
        v3 = p3a_r[...] + p3b_r[...]
        hnb = jnp.where(v3 >= 0, v3, a3_r[0, 0] * v3)
        t2 = jax.lax.dot_general(hnb, wd2_r[...],
                                 (((1,), (1,)), ((), ())),
                                 preferred_element_type=jnp.float32)
        t2_r[...] = t2
        mp_r[...] = (jnp.sum(u1_r[...] * h2, axis=-1)[:, None]
                     + bd1_r[0, 0] + sb1_r[...])
        lp_r[...] = (jnp.sum(h2 * t2, axis=-1)[:, None]
                     + bd2_r[0, 0] + sb1_r[...])

    return pl.pallas_call(
        body,
        grid=(NRB,),
        in_specs=[
            pl.BlockSpec((RB, D), lambda i: (i, 0)),
            pl.BlockSpec((RB, D), lambda i: (i, 0)),
            pl.BlockSpec((1, D), lambda i: (0, 0)),
            pl.BlockSpec((1, 1), lambda i: (0, 0), memory_space=pltpu.SMEM),
            pl.BlockSpec((RB, D), lambda i: (i, 0)),
            pl.BlockSpec((RB, D), lambda i: (i, 0)),
            pl.BlockSpec((1, 1), lambda i: (0, 0), memory_space=pltpu.SMEM),
            pl.BlockSpec((D, D), lambda i: (0, 0)),
            pl.BlockSpec((RB, D), lambda i: (i, 0)),
            pl.BlockSpec((RB, 1), lambda i: (i, 0)),
            pl.BlockSpec((1, 1), lambda i: (0, 0), memory_space=pltpu.SMEM),
            pl.BlockSpec((1, 1), lambda i: (0, 0), memory_space=pltpu.SMEM),
        ],
        out_specs=[
            pl.BlockSpec((RB, D), lambda i: (i, 0)),
            pl.BlockSpec((RB, D), lambda i: (i, 0)),
            pl.BlockSpec((RB, 1), lambda i: (i, 0)),
            pl.BlockSpec((RB, 1), lambda i: (i, 0)),
        ],
        out_shape=[
            jax.ShapeDtypeStruct((N, D), jnp.float32),
            jax.ShapeDtypeStruct((N, D), jnp.float32),
            jax.ShapeDtypeStruct((N, 1), jnp.float32),
            jax.ShapeDtypeStruct((N, 1), jnp.float32),
        ],
    )(p2[0], p2[1], b2, a2, p3[0], p3[1], a3, Wd2, u1, samp_bias1, bd1, bd2)


def _neg_dots(g1, g2, h2, t2, samp_bias2, bd1, bd2):
    def body(g1_r, g2_r, h2_r, t2_r, sb2_r, bd1_r, bd2_r, o1_r, o2_r):
        h2 = h2_r[...][None]
        t2 = t2_r[...][None]
        o1_r[...] = (jnp.sum(g1_r[...] * h2, axis=-1).T
                     + bd1_r[0, 0] + sb2_r[...])
        o2_r[...] = (jnp.sum(g2_r[...] * t2, axis=-1).T
                     + bd2_r[0, 0] + sb2_r[...])

    return pl.pallas_call(
        body,
        grid=(NRB,),
        in_specs=[
            pl.BlockSpec((NEG, RB, D), lambda i: (0, i, 0)),
            pl.BlockSpec((NEG, RB, D), lambda i: (0, i, 0)),
            pl.BlockSpec((RB, D), lambda i: (i, 0)),
            pl.BlockSpec((RB, D), lambda i: (i, 0)),
            pl.BlockSpec((RB, NEG), lambda i: (i, 0)),
            pl.BlockSpec((1, 1), lambda i: (0, 0), memory_space=pltpu.SMEM),
            pl.BlockSpec((1, 1), lambda i: (0, 0), memory_space=pltpu.SMEM),
        ],
        out_specs=[
            pl.BlockSpec((RB, NEG), lambda i: (i, 0)),
            pl.BlockSpec((RB, NEG), lambda i: (i, 0)),
        ],
        out_shape=[
            jax.ShapeDtypeStruct((N, NEG), jnp.float32),
            jax.ShapeDtypeStruct((N, NEG), jnp.float32),
        ],
    )(g1, g2, h2, t2, samp_bias2, bd1, bd2)


def _adj_rebuild(h2):
    def body(a_r, b_r, o_r):
        prod = jax.lax.dot_general(a_r[...], b_r[...],
                                   (((1,), (1,)), ((), ())),
                                   preferred_element_type=jnp.float32)
        o_r[...] = jax.nn.sigmoid(prod)

    bk = 1024
    nbk = (N + bk - 1) // bk
    return pl.pallas_call(
        body,
        grid=(nbk, nbk),
        in_specs=[
            pl.BlockSpec((bk, D), lambda i, j: (i, 0)),
            pl.BlockSpec((bk, D), lambda i, j: (j, 0)),
        ],
        out_specs=pl.BlockSpec((bk, bk), lambda i, j: (i, j)),
        out_shape=jax.ShapeDtypeStruct((N, N), jnp.float32),
    )(h2, h2)


def kernel(seq1, adj_index, adj_weight, adj_ori_index, adj_ori_weight,
           neg_num, neg_samples, samp_bias1, samp_bias2, W1, b1, a1, W2, b2,
           a2, a3, Wd1, bd1, Wd2, bd2):
    x = seq1[0]
    zeros = jnp.zeros((N, D), jnp.float32)

    def edges3(idx, w):
        src = idx[1].astype(jnp.int32).reshape(NW, ECPW, 1, EC)
        dst = idx[0].astype(jnp.int32).reshape(NW, ECPW, 1, EC)
        sd = jnp.concatenate([src, dst], axis=2)
        return sd, w.reshape(NW, ECPW * EC)

    sd1, w1e = edges3(adj_index, adj_weight)
    sd3, w3e = edges3(adj_ori_index, adj_ori_weight)

    b1r = b1.reshape(1, D)
    b2r = b2.reshape(1, D)
    a1r = a1.reshape(1, 1)
    a2r = a2.reshape(1, 1)
    a3r = a3.reshape(1, 1)
    bd1r = bd1.reshape(1, 1)
    bd2r = bd2.reshape(1, 1)

    sb1t = samp_bias1.reshape(N, 1)
    sb2t = samp_bias2.T

    h_w, u1 = _proj2(x, W1, Wd1)
    p1 = _spmm_partials(sd1, w1e, h_w, zeros)
    p3 = _spmm_partials(sd3, w3e, h_w, zeros)
    fts2 = _gcn1_tail(p1, b1r, a1r, W2)
    p2 = _spmm_partials(sd1, w1e, fts2, zeros)
    h2, t2, mp_t, lp_t = _stage2(
        p2, b2r, a2r, p3, a3r, Wd2, u1, sb1t, bd1r, bd2r)
    res_mi_pos = mp_t.reshape(1, N)
    res_local_pos = lp_t.reshape(1, N)

    neg_flat = neg_samples.astype(jnp.int32).reshape(-1)
    neg_pad = jnp.concatenate(
        [neg_flat, jnp.zeros((NEG_PAD - NEG * N,), jnp.int32)])
    idx3 = neg_pad.reshape(NW, GCPW, GC)
    g1f, g2f = _gather_rows2(idx3, u1, h2)
    g1 = g1f[:NEG * N].reshape(NEG, N, D)
    g2 = g2f[:NEG * N].reshape(NEG, N, D)

    mn_t, ln_t = _neg_dots(g1, g2, h2, t2, sb2t, bd1r, bd2r)
    res_mi_neg = mn_t.T
    res_local_neg = ln_t.T

    adj_rebuilt = _adj_rebuild(h2)
    return (res_mi_pos, res_mi_neg, res_local_pos, res_local_neg, adj_rebuilt)


# speedup vs baseline: 3.9652x; 1.2772x over previous
import functools

import jax
import jax.numpy as jnp
from jax import lax
from jax.experimental import pallas as pl
from jax.experimental.pallas import tpu as pltpu
from jax.experimental.pallas import tpu_sc as plsc

N = 10000
E = 320000
D = 128
NEG = 5
NC, NS = 2, 16
NW = NC * NS
EC = 80
ECPW = E // (NW * EC)
ROWS_PT = 624
ROWS_TAIL = N - NS * ROWS_PT

GC = 112
GCPW = 14
NEG_PAD = NW * GCPW * GC


def _sc_mesh():
    return plsc.VectorSubcoreMesh(core_axis_name="c", subcore_axis_name="s")


def _spmm_partials(sd3, w3, feats, zeros):
    @functools.partial(
        pl.kernel,
        mesh=_sc_mesh(),
        out_type=jax.ShapeDtypeStruct((NC, N, D), jnp.float32),
        scratch_types=[
            pltpu.VMEM((2, EC), jnp.int32),
            pltpu.VMEM((2, EC), jnp.int32),
            pltpu.VMEM((ECPW * EC,), jnp.float32),
            pltpu.VMEM((EC, D), jnp.float32),
            pltpu.VMEM((EC, D), jnp.float32),
            pltpu.VMEM_SHARED((N, D), jnp.float32),
            pltpu.SemaphoreType.DMA,
            pltpu.SemaphoreType.DMA,
        ],
    )
    def k(sd_h, w_h, x_h, z_h, out_h, sd_v0, sd_v1, w_v, rows_v0, rows_v1,
          acc_s, sem0, sem1):
        c = lax.axis_index("c")
        s = lax.axis_index("s")
        wid = s * NC + c
        pltpu.sync_copy(z_h.at[pl.ds(s * ROWS_PT, ROWS_PT)],
                        acc_s.at[pl.ds(s * ROWS_PT, ROWS_PT)])

        @pl.when(s == NS - 1)
        def _zero_tail():
            pltpu.sync_copy(z_h.at[pl.ds(NS * ROWS_PT, ROWS_TAIL)],
                            acc_s.at[pl.ds(NS * ROWS_PT, ROWS_TAIL)])
        pltpu.sync_copy(w_h.at[wid], w_v)
        plsc.subcore_barrier()

        dnums = lax.GatherDimensionNumbers(
            offset_dims=(), collapsed_slice_dims=(0,), start_index_map=(0,))
        sd = (sd_v0, sd_v1)
        rows = (rows_v0, rows_v1)
        sems = (sem0, sem1)

        def process(t, bb, nb):
            pltpu.make_async_copy(x_h.at[sd[bb].at[0]], rows[bb],
                                  sems[bb]).wait()

            @pl.when(t + 1 < ECPW)
            def _prefetch():
                pltpu.sync_copy(sd_h.at[wid, t + 1], sd[nb])
                pltpu.async_copy(x_h.at[sd[nb].at[0]], rows[nb], sems[nb])

            def grp_body(gi, carry2):
                wv = w_v[pl.ds(t * EC + gi * 16, 16)]
                for j in range(16):
                    wb = lax.gather(
                        wv, jnp.full((16, 1), j, jnp.int32), dnums, (1,),
                        mode=lax.GatherScatterMode.PROMISE_IN_BOUNDS)
                    e = gi * 16 + j
                    for g in range(D // 16):
                        rows[bb][e, pl.ds(g * 16, 16)] = (
                            rows[bb][e, pl.ds(g * 16, 16)] * wb)
                return carry2

            lax.fori_loop(0, EC // 16, grp_body, 0)
            pltpu.sync_copy(rows[bb], acc_s.at[sd[bb].at[1]], add=True)

        pltpu.sync_copy(sd_h.at[wid, 0], sd_v0)
        pltpu.async_copy(x_h.at[sd_v0.at[0]], rows_v0, sem0)

        def pair_body(p, carry):
            process(2 * p, 0, 1)
            process(2 * p + 1, 1, 0)
            return carry

        lax.fori_loop(0, ECPW // 2, pair_body, 0)
        if ECPW % 2:
            process(ECPW - 1, 0, 1)
        plsc.subcore_barrier()
        pltpu.sync_copy(acc_s.at[pl.ds(s * ROWS_PT, ROWS_PT)],
                        out_h.at[c, pl.ds(s * ROWS_PT, ROWS_PT)])

        @pl.when(s == NS - 1)
        def _write_tail():
            pltpu.sync_copy(acc_s.at[pl.ds(NS * ROWS_PT, ROWS_TAIL)],
                            out_h.at[c, pl.ds(NS * ROWS_PT, ROWS_TAIL)])

    return k(sd3, w3, feats, zeros)


def _gather_rows2(idx3, tab_a, tab_b):
    @functools.partial(
        pl.kernel,
        mesh=_sc_mesh(),
        out_type=(jax.ShapeDtypeStruct((NEG_PAD, D), jnp.float32),
                  jax.ShapeDtypeStruct((NEG_PAD, D), jnp.float32)),
        scratch_types=[
            pltpu.VMEM((GCPW, GC), jnp.int32),
            pltpu.VMEM((GC, D), jnp.float32),
            pltpu.VMEM((GC, D), jnp.float32),
            pltpu.SemaphoreType.DMA,
            pltpu.SemaphoreType.DMA,
        ],
    )
    def k(idx_h, a_h, b_h, oa_h, ob_h, idx_v, buf_a, buf_b, sem_a, sem_b):
        c = lax.axis_index("c")
        s = lax.axis_index("s")
        wid = s * NC + c
        base = wid * (GCPW * GC)
        pltpu.sync_copy(idx_h.at[wid], idx_v)

        def body(t, carry):
            cpa = pltpu.async_copy(a_h.at[idx_v.at[t]], buf_a, sem_a)
            cpb = pltpu.async_copy(b_h.at[idx_v.at[t]], buf_b, sem_b)
            cpa.wait()
            pltpu.sync_copy(buf_a, oa_h.at[pl.ds(base + t * GC, GC)])
            cpb.wait()
            pltpu.sync_copy(buf_b, ob_h.at[pl.ds(base + t * GC, GC)])
            return carry

        lax.fori_loop(0, GCPW, body, 0)

    return k(idx3, tab_a, tab_b)


RB = 1000
NRB = N // RB


def _proj2(x, W1, Wd1):
    def body(x_r, w1_r, wd1_r, o1_r, o2_r):
        xb = x_r[...]
        o1_r[...] = jnp.dot(xb, w1_r[...], preferred_element_type=jnp.float32)
        o2_r[...] = jnp.dot(xb, wd1_r[...], preferred_element_type=jnp.float32)

    return pl.pallas_call(
        body,
        grid=(NRB,),
        in_specs=[
            pl.BlockSpec((RB, D), lambda i: (i, 0)),
            pl.BlockSpec((D, D), lambda i: (0, 0)),
            pl.BlockSpec((D, D), lambda i: (0, 0)),
        ],
        out_specs=[
            pl.BlockSpec((RB, D), lambda i: (i, 0)),
            pl.BlockSpec((RB, D), lambda i: (i, 0)),
        ],
        out_shape=[
            jax.ShapeDtypeStruct((N, D), jnp.float32),
            jax.ShapeDtypeStruct((N, D), jnp.float32),
        ],
    )(x, W1, Wd1)


def _gcn1_tail(p1, b1, a1, W2):
    def body(pa_r, pb_r, b_r, a_r, w2_r, o_r):
        v = pa_r[...] + pb_r[...] + b_r[...]
        h = jnp.where(v >= 0, v, a_r[0, 0] * v)
        o_r[...] = jnp.dot(h, w2_r[...], preferred_element_type=jnp.float32)

    return pl.pallas_call(
        body,
        grid=(NRB,),
        in_specs=[
            pl.BlockSpec((RB, D), lambda i: (i, 0)),
            pl.BlockSpec((RB, D), lambda i: (i, 0)),
            pl.BlockSpec((1, D), lambda i: (0, 0)),
            pl.BlockSpec((1, 1), lambda i: (0, 0), memory_space=pltpu.SMEM),
            pl.BlockSpec((D, D), lambda i: (0, 0)),
        ],
        out_specs=pl.BlockSpec((RB, D), lambda i: (i, 0)),
        out_shape=jax.ShapeDtypeStruct((N, D), jnp.float32),
    )(p1[0], p1[1], b1, a1, W2)


def _stage2(p2, b2, a2, p3, a3, Wd2, u1, samp_bias1, bd1, bd2):
    def body(p2a_r, p2b_r, b2_r, a2_r, p3a_r, p3b_r, a3_r, wd2_r, u1_r,
             sb1_r, bd1_r, bd2_r, h2_r, t2_r, mp_r, lp_r):
        v2 = p2a_r[...] + p2b_r[...] + b2_r[...]
        h2 = jnp.where(v2 >= 0, v2, a2_r[0, 0] * v2)
        h2_r[...] = h2
        v3 = p3a_r[...] + p3b_r[...]
        hnb = jnp.where(v3 >= 0, v3, a3_r[0, 0] * v3)
        t2 = jax.lax.dot_general(hnb, wd2_r[...],
                                 (((1,), (1,)), ((), ())),
                                 preferred_element_type=jnp.float32)
        t2_r[...] = t2
        mp_r[...] = (jnp.sum(u1_r[...] * h2, axis=-1)[:, None]
                     + bd1_r[0, 0] + sb1_r[...])
        lp_r[...] = (jnp.sum(h2 * t2, axis=-1)[:, None]
                     + bd2_r[0, 0] + sb1_r[...])

    return pl.pallas_call(
        body,
        grid=(NRB,),
        in_specs=[
            pl.BlockSpec((RB, D), lambda i: (i, 0)),
            pl.BlockSpec((RB, D), lambda i: (i, 0)),
            pl.BlockSpec((1, D), lambda i: (0, 0)),
            pl.BlockSpec((1, 1), lambda i: (0, 0), memory_space=pltpu.SMEM),
            pl.BlockSpec((RB, D), lambda i: (i, 0)),
            pl.BlockSpec((RB, D), lambda i: (i, 0)),
            pl.BlockSpec((1, 1), lambda i: (0, 0), memory_space=pltpu.SMEM),
            pl.BlockSpec((D, D), lambda i: (0, 0)),
            pl.BlockSpec((RB, D), lambda i: (i, 0)),
            pl.BlockSpec((RB, 1), lambda i: (i, 0)),
            pl.BlockSpec((1, 1), lambda i: (0, 0), memory_space=pltpu.SMEM),
            pl.BlockSpec((1, 1), lambda i: (0, 0), memory_space=pltpu.SMEM),
        ],
        out_specs=[
            pl.BlockSpec((RB, D), lambda i: (i, 0)),
            pl.BlockSpec((RB, D), lambda i: (i, 0)),
            pl.BlockSpec((RB, 1), lambda i: (i, 0)),
            pl.BlockSpec((RB, 1), lambda i: (i, 0)),
        ],
        out_shape=[
            jax.ShapeDtypeStruct((N, D), jnp.float32),
            jax.ShapeDtypeStruct((N, D), jnp.float32),
            jax.ShapeDtypeStruct((N, 1), jnp.float32),
            jax.ShapeDtypeStruct((N, 1), jnp.float32),
        ],
    )(p2[0], p2[1], b2, a2, p3[0], p3[1], a3, Wd2, u1, samp_bias1, bd1, bd2)


def _neg_dots(g1, g2, h2, t2, samp_bias2, bd1, bd2):
    def body(g1_r, g2_r, h2_r, t2_r, sb2_r, bd1_r, bd2_r, o1_r, o2_r):
        h2 = h2_r[...][None]
        t2 = t2_r[...][None]
        o1_r[...] = (jnp.sum(g1_r[...] * h2, axis=-1).T
                     + bd1_r[0, 0] + sb2_r[...])
        o2_r[...] = (jnp.sum(g2_r[...] * t2, axis=-1).T
                     + bd2_r[0, 0] + sb2_r[...])

    return pl.pallas_call(
        body,
        grid=(NRB,),
        in_specs=[
            pl.BlockSpec((NEG, RB, D), lambda i: (0, i, 0)),
            pl.BlockSpec((NEG, RB, D), lambda i: (0, i, 0)),
            pl.BlockSpec((RB, D), lambda i: (i, 0)),
            pl.BlockSpec((RB, D), lambda i: (i, 0)),
            pl.BlockSpec((RB, NEG), lambda i: (i, 0)),
            pl.BlockSpec((1, 1), lambda i: (0, 0), memory_space=pltpu.SMEM),
            pl.BlockSpec((1, 1), lambda i: (0, 0), memory_space=pltpu.SMEM),
        ],
        out_specs=[
            pl.BlockSpec((RB, NEG), lambda i: (i, 0)),
            pl.BlockSpec((RB, NEG), lambda i: (i, 0)),
        ],
        out_shape=[
            jax.ShapeDtypeStruct((N, NEG), jnp.float32),
            jax.ShapeDtypeStruct((N, NEG), jnp.float32),
        ],
    )(g1, g2, h2, t2, samp_bias2, bd1, bd2)


def _adj_rebuild(h2):
    def body(a_r, b_r, o_r):
        prod = jax.lax.dot_general(a_r[...], b_r[...],
                                   (((1,), (1,)), ((), ())),
                                   preferred_element_type=jnp.float32)
        o_r[...] = jax.nn.sigmoid(prod)

    bk = 1024
    nbk = (N + bk - 1) // bk
    return pl.pallas_call(
        body,
        grid=(nbk, nbk),
        in_specs=[
            pl.BlockSpec((bk, D), lambda i, j: (i, 0)),
            pl.BlockSpec((bk, D), lambda i, j: (j, 0)),
        ],
        out_specs=pl.BlockSpec((bk, bk), lambda i, j: (i, j)),
        out_shape=jax.ShapeDtypeStruct((N, N), jnp.float32),
    )(h2, h2)


def kernel(seq1, adj_index, adj_weight, adj_ori_index, adj_ori_weight,
           neg_num, neg_samples, samp_bias1, samp_bias2, W1, b1, a1, W2, b2,
           a2, a3, Wd1, bd1, Wd2, bd2):
    x = seq1[0]
    zeros = jnp.zeros((N, D), jnp.float32)

    def edges3(idx, w):
        src = idx[1].astype(jnp.int32).reshape(NW, ECPW, 1, EC)
        dst = idx[0].astype(jnp.int32).reshape(NW, ECPW, 1, EC)
        sd = jnp.concatenate([src, dst], axis=2)
        return sd, w.reshape(NW, ECPW * EC)

    sd1, w1e = edges3(adj_index, adj_weight)
    sd3, w3e = edges3(adj_ori_index, adj_ori_weight)

    b1r = b1.reshape(1, D)
    b2r = b2.reshape(1, D)
    a1r = a1.reshape(1, 1)
    a2r = a2.reshape(1, 1)
    a3r = a3.reshape(1, 1)
    bd1r = bd1.reshape(1, 1)
    bd2r = bd2.reshape(1, 1)

    sb1t = samp_bias1.reshape(N, 1)
    sb2t = samp_bias2.T

    h_w, u1 = _proj2(x, W1, Wd1)
    p1 = _spmm_partials(sd1, w1e, h_w, zeros)
    p3 = _spmm_partials(sd3, w3e, h_w, zeros)
    fts2 = _gcn1_tail(p1, b1r, a1r, W2)
    p2 = _spmm_partials(sd1, w1e, fts2, zeros)
    h2, t2, mp_t, lp_t = _stage2(
        p2, b2r, a2r, p3, a3r, Wd2, u1, sb1t, bd1r, bd2r)
    res_mi_pos = mp_t.reshape(1, N)
    res_local_pos = lp_t.reshape(1, N)

    neg_flat = neg_samples.astype(jnp.int32).reshape(-1)
    neg_pad = jnp.concatenate(
        [neg_flat, jnp.zeros((NEG_PAD - NEG * N,), jnp.int32)])
    idx3 = neg_pad.reshape(NW, GCPW, GC)
    g1f, g2f = _gather_rows2(idx3, u1, h2)
    g1 = g1f[:NEG * N].reshape(NEG, N, D)
    g2 = g2f[:NEG * N].reshape(NEG, N, D)

    mn_t, ln_t = _neg_dots(g1, g2, h2, t2, sb2t, bd1r, bd2r)
    res_mi_neg = mn_t.T
    res_local_neg = ln_t.T

    adj_rebuilt = _adj_rebuild(h2)
    return (res_mi_pos, res_mi_neg, res_local_pos, res_local_neg, adj_rebuilt)


# speedup vs baseline: 4.7947x; 1.2092x over previous
import functools

import jax
import jax.numpy as jnp
from jax import lax
from jax.experimental import pallas as pl
from jax.experimental.pallas import tpu as pltpu
from jax.experimental.pallas import tpu_sc as plsc

N = 10000
E = 320000
D = 128
NEG = 5
NC, NS = 2, 16
NW = NC * NS
EC = 80
ECPW = E // (NW * EC)
ROWS_PT = 624
ROWS_TAIL = N - NS * ROWS_PT

GC = 112
GCPW = 14
NEG_PAD = NW * GCPW * GC


def _sc_mesh():
    return plsc.VectorSubcoreMesh(core_axis_name="c", subcore_axis_name="s")


def _spmm_partials(src2, dst4, w2, feats, zeros):
    epw = E // NW

    @functools.partial(
        pl.kernel,
        mesh=_sc_mesh(),
        out_type=jax.ShapeDtypeStruct((NC, N, D), jnp.float32),
        scratch_types=[
            pltpu.VMEM((epw,), jnp.int32),
            pltpu.VMEM((epw,), jnp.float32),
            pltpu.VMEM((1, EC), jnp.int32),
            pltpu.VMEM((1, EC), jnp.int32),
            pltpu.VMEM((EC, D), jnp.float32),
            pltpu.VMEM((EC, D), jnp.float32),
            pltpu.VMEM_SHARED((N, D), jnp.float32),
            pltpu.SemaphoreType.DMA,
            pltpu.SemaphoreType.DMA,
            pltpu.SemaphoreType.DMA,
            pltpu.SemaphoreType.DMA,
        ],
    )
    def k(src_h, dst_h, w_h, x_h, z_h, out_h, src_v, w_v, dst_v0, dst_v1,
          rows_v0, rows_v1, acc_s, gsem0, gsem1, dsem0, dsem1):
        c = lax.axis_index("c")
        s = lax.axis_index("s")
        wid = s * NC + c
        pltpu.sync_copy(z_h.at[pl.ds(s * ROWS_PT, ROWS_PT)],
                        acc_s.at[pl.ds(s * ROWS_PT, ROWS_PT)])

        @pl.when(s == NS - 1)
        def _zero_tail():
            pltpu.sync_copy(z_h.at[pl.ds(NS * ROWS_PT, ROWS_TAIL)],
                            acc_s.at[pl.ds(NS * ROWS_PT, ROWS_TAIL)])
        pltpu.sync_copy(src_h.at[wid], src_v)
        pltpu.sync_copy(w_h.at[wid], w_v)
        plsc.subcore_barrier()

        dnums = lax.GatherDimensionNumbers(
            offset_dims=(), collapsed_slice_dims=(0,), start_index_map=(0,))
        rows = (rows_v0, rows_v1)
        dst = (dst_v0, dst_v1)
        gsems = (gsem0, gsem1)
        dsems = (dsem0, dsem1)

        def process(t, bb, nb):
            pltpu.make_async_copy(x_h.at[src_v.at[pl.ds(t * EC, EC)]],
                                  rows[bb], gsems[bb]).wait()

            @pl.when(t + 1 < ECPW)
            def _prefetch():
                pltpu.async_copy(
                    x_h.at[src_v.at[pl.ds((t + 1) * EC, EC)]], rows[nb],
                    gsems[nb])
                pltpu.async_copy(dst_h.at[wid, t + 1], dst[nb], dsems[nb])

            @plsc.parallel_loop(0, EC // 16)
            def grp_body(gi):
                wv = w_v[pl.ds(t * EC + gi * 16, 16)]
                for j in range(16):
                    wb = lax.gather(
                        wv, jnp.full((16, 1), j, jnp.int32), dnums, (1,),
                        mode=lax.GatherScatterMode.PROMISE_IN_BOUNDS)
                    e = gi * 16 + j
                    for g in range(D // 16):
                        rows[bb][e, pl.ds(g * 16, 16)] = (
                            rows[bb][e, pl.ds(g * 16, 16)] * wb)

            pltpu.make_async_copy(dst_h.at[wid, t], dst[bb],
                                  dsems[bb]).wait()
            pltpu.sync_copy(rows[bb], acc_s.at[dst[bb].at[0]], add=True)

        pltpu.async_copy(dst_h.at[wid, 0], dst_v0, dsem0)
        pltpu.async_copy(x_h.at[src_v.at[pl.ds(0, EC)]], rows_v0, gsem0)

        def pair_body(p, carry):
            process(2 * p, 0, 1)
            process(2 * p + 1, 1, 0)
            return carry

        lax.fori_loop(0, ECPW // 2, pair_body, 0)
        if ECPW % 2:
            process(ECPW - 1, 0, 1)
        plsc.subcore_barrier()
        pltpu.sync_copy(acc_s.at[pl.ds(s * ROWS_PT, ROWS_PT)],
                        out_h.at[c, pl.ds(s * ROWS_PT, ROWS_PT)])

        @pl.when(s == NS - 1)
        def _write_tail():
            pltpu.sync_copy(acc_s.at[pl.ds(NS * ROWS_PT, ROWS_TAIL)],
                            out_h.at[c, pl.ds(NS * ROWS_PT, ROWS_TAIL)])

    return k(src2, dst4, w2, feats, zeros)


def _gather_rows2(idx3, tab_a, tab_b):
    @functools.partial(
        pl.kernel,
        mesh=_sc_mesh(),
        out_type=(jax.ShapeDtypeStruct((NEG_PAD, D), jnp.float32),
                  jax.ShapeDtypeStruct((NEG_PAD, D), jnp.float32)),
        scratch_types=[
            pltpu.VMEM((GCPW, GC), jnp.int32),
            pltpu.VMEM((GC, D), jnp.float32),
            pltpu.VMEM((GC, D), jnp.float32),
            pltpu.VMEM((GC, D), jnp.float32),
            pltpu.VMEM((GC, D), jnp.float32),
            pltpu.SemaphoreType.DMA,
            pltpu.SemaphoreType.DMA,
            pltpu.SemaphoreType.DMA,
            pltpu.SemaphoreType.DMA,
        ],
    )
    def k(idx_h, a_h, b_h, oa_h, ob_h, idx_v, buf_a0, buf_a1, buf_b0,
          buf_b1, sem_a0, sem_a1, sem_b0, sem_b1):
        c = lax.axis_index("c")
        s = lax.axis_index("s")
        wid = s * NC + c
        base = wid * (GCPW * GC)
        pltpu.sync_copy(idx_h.at[wid], idx_v)
        bufs_a = (buf_a0, buf_a1)
        bufs_b = (buf_b0, buf_b1)
        sems_a = (sem_a0, sem_a1)
        sems_b = (sem_b0, sem_b1)

        def process(t, bb, nb):
            pltpu.make_async_copy(a_h.at[idx_v.at[t]], bufs_a[bb],
                                  sems_a[bb]).wait()

            @pl.when(t + 1 < GCPW)
            def _pf_a():
                pltpu.async_copy(a_h.at[idx_v.at[t + 1]], bufs_a[nb],
                                 sems_a[nb])
            pltpu.sync_copy(bufs_a[bb], oa_h.at[pl.ds(base + t * GC, GC)])
            pltpu.make_async_copy(b_h.at[idx_v.at[t]], bufs_b[bb],
                                  sems_b[bb]).wait()

            @pl.when(t + 1 < GCPW)
            def _pf_b():
                pltpu.async_copy(b_h.at[idx_v.at[t + 1]], bufs_b[nb],
                                 sems_b[nb])
            pltpu.sync_copy(bufs_b[bb], ob_h.at[pl.ds(base + t * GC, GC)])

        pltpu.async_copy(a_h.at[idx_v.at[0]], buf_a0, sem_a0)
        pltpu.async_copy(b_h.at[idx_v.at[0]], buf_b0, sem_b0)

        def pair_body(p, carry):
            process(2 * p, 0, 1)
            process(2 * p + 1, 1, 0)
            return carry

        lax.fori_loop(0, GCPW // 2, pair_body, 0)

    return k(idx3, tab_a, tab_b)


RB = 1000
NRB = N // RB


def _proj2(x, W1, Wd1):
    def body(x_r, w1_r, wd1_r, o1_r, o2_r):
        xb = x_r[...]
        o1_r[...] = jnp.dot(xb, w1_r[...], preferred_element_type=jnp.float32)
        o2_r[...] = jnp.dot(xb, wd1_r[...], preferred_element_type=jnp.float32)

    return pl.pallas_call(
        body,
        grid=(NRB,),
        in_specs=[
            pl.BlockSpec((RB, D), lambda i: (i, 0)),
            pl.BlockSpec((D, D), lambda i: (0, 0)),
            pl.BlockSpec((D, D), lambda i: (0, 0)),
        ],
        out_specs=[
            pl.BlockSpec((RB, D), lambda i: (i, 0)),
            pl.BlockSpec((RB, D), lambda i: (i, 0)),
        ],
        out_shape=[
            jax.ShapeDtypeStruct((N, D), jnp.float32),
            jax.ShapeDtypeStruct((N, D), jnp.float32),
        ],
    )(x, W1, Wd1)


def _gcn1_tail(p1, b1, a1, W2):
    def body(pa_r, pb_r, b_r, a_r, w2_r, o_r):
        v = pa_r[...] + pb_r[...] + b_r[...]
        h = jnp.where(v >= 0, v, a_r[0, 0] * v)
        o_r[...] = jnp.dot(h, w2_r[...], preferred_element_type=jnp.float32)

    return pl.pallas_call(
        body,
        grid=(NRB,),
        in_specs=[
            pl.BlockSpec((RB, D), lambda i: (i, 0)),
            pl.BlockSpec((RB, D), lambda i: (i, 0)),
            pl.BlockSpec((1, D), lambda i: (0, 0)),
            pl.BlockSpec((1, 1), lambda i: (0, 0), memory_space=pltpu.SMEM),
            pl.BlockSpec((D, D), lambda i: (0, 0)),
        ],
        out_specs=pl.BlockSpec((RB, D), lambda i: (i, 0)),
        out_shape=jax.ShapeDtypeStruct((N, D), jnp.float32),
    )(p1[0], p1[1], b1, a1, W2)


def _stage2(p2, b2, a2, p3, a3, Wd2, u1, samp_bias1, bd1, bd2):
    def body(p2a_r, p2b_r, b2_r, a2_r, p3a_r, p3b_r, a3_r, wd2_r, u1_r,
             sb1_r, bd1_r, bd2_r, h2_r, t2_r, mp_r, lp_r):
        v2 = p2a_r[...] + p2b_r[...] + b2_r[...]
        h2 = jnp.where(v2 >= 0, v2, a2_r[0, 0] * v2)
        h2_r[...] = h2
        v3 = p3a_r[...] + p3b_r[...]
        hnb = jnp.where(v3 >= 0, v3, a3_r[0, 0] * v3)
        t2 = jax.lax.dot_general(hnb, wd2_r[...],
                                 (((1,), (1,)), ((), ())),
                                 preferred_element_type=jnp.float32)
        t2_r[...] = t2
        mp_r[...] = (jnp.sum(u1_r[...] * h2, axis=-1)[:, None]
                     + bd1_r[0, 0] + sb1_r[...])
        lp_r[...] = (jnp.sum(h2 * t2, axis=-1)[:, None]
                     + bd2_r[0, 0] + sb1_r[...])

    return pl.pallas_call(
        body,
        grid=(NRB,),
        in_specs=[
            pl.BlockSpec((RB, D), lambda i: (i, 0)),
            pl.BlockSpec((RB, D), lambda i: (i, 0)),
            pl.BlockSpec((1, D), lambda i: (0, 0)),
            pl.BlockSpec((1, 1), lambda i: (0, 0), memory_space=pltpu.SMEM),
            pl.BlockSpec((RB, D), lambda i: (i, 0)),
            pl.BlockSpec((RB, D), lambda i: (i, 0)),
            pl.BlockSpec((1, 1), lambda i: (0, 0), memory_space=pltpu.SMEM),
            pl.BlockSpec((D, D), lambda i: (0, 0)),
            pl.BlockSpec((RB, D), lambda i: (i, 0)),
            pl.BlockSpec((RB, 1), lambda i: (i, 0)),
            pl.BlockSpec((1, 1), lambda i: (0, 0), memory_space=pltpu.SMEM),
            pl.BlockSpec((1, 1), lambda i: (0, 0), memory_space=pltpu.SMEM),
        ],
        out_specs=[
            pl.BlockSpec((RB, D), lambda i: (i, 0)),
            pl.BlockSpec((RB, D), lambda i: (i, 0)),
            pl.BlockSpec((RB, 1), lambda i: (i, 0)),
            pl.BlockSpec((RB, 1), lambda i: (i, 0)),
        ],
        out_shape=[
            jax.ShapeDtypeStruct((N, D), jnp.float32),
            jax.ShapeDtypeStruct((N, D), jnp.float32),
            jax.ShapeDtypeStruct((N, 1), jnp.float32),
            jax.ShapeDtypeStruct((N, 1), jnp.float32),
        ],
    )(p2[0], p2[1], b2, a2, p3[0], p3[1], a3, Wd2, u1, samp_bias1, bd1, bd2)


def _neg_dots(g1, g2, h2, t2, samp_bias2, bd1, bd2):
    def body(g1_r, g2_r, h2_r, t2_r, sb2_r, bd1_r, bd2_r, o1_r, o2_r):
        h2 = h2_r[...][None]
        t2 = t2_r[...][None]
        o1_r[...] = (jnp.sum(g1_r[...] * h2, axis=-1).T
                     + bd1_r[0, 0] + sb2_r[...])
        o2_r[...] = (jnp.sum(g2_r[...] * t2, axis=-1).T
                     + bd2_r[0, 0] + sb2_r[...])

    return pl.pallas_call(
        body,
        grid=(NRB,),
        in_specs=[
            pl.BlockSpec((NEG, RB, D), lambda i: (0, i, 0)),
            pl.BlockSpec((NEG, RB, D), lambda i: (0, i, 0)),
            pl.BlockSpec((RB, D), lambda i: (i, 0)),
            pl.BlockSpec((RB, D), lambda i: (i, 0)),
            pl.BlockSpec((RB, NEG), lambda i: (i, 0)),
            pl.BlockSpec((1, 1), lambda i: (0, 0), memory_space=pltpu.SMEM),
            pl.BlockSpec((1, 1), lambda i: (0, 0), memory_space=pltpu.SMEM),
        ],
        out_specs=[
            pl.BlockSpec((RB, NEG), lambda i: (i, 0)),
            pl.BlockSpec((RB, NEG), lambda i: (i, 0)),
        ],
        out_shape=[
            jax.ShapeDtypeStruct((N, NEG), jnp.float32),
            jax.ShapeDtypeStruct((N, NEG), jnp.float32),
        ],
    )(g1, g2, h2, t2, samp_bias2, bd1, bd2)


def _adj_rebuild(h2):
    def body(a_r, b_r, o_r):
        prod = jax.lax.dot_general(a_r[...], b_r[...],
                                   (((1,), (1,)), ((), ())),
                                   preferred_element_type=jnp.float32)
        o_r[...] = jax.nn.sigmoid(prod)

    bk = 1024
    nbk = (N + bk - 1) // bk
    return pl.pallas_call(
        body,
        grid=(nbk, nbk),
        in_specs=[
            pl.BlockSpec((bk, D), lambda i, j: (i, 0)),
            pl.BlockSpec((bk, D), lambda i, j: (j, 0)),
        ],
        out_specs=pl.BlockSpec((bk, bk), lambda i, j: (i, j)),
        out_shape=jax.ShapeDtypeStruct((N, N), jnp.float32),
    )(h2, h2)


def kernel(seq1, adj_index, adj_weight, adj_ori_index, adj_ori_weight,
           neg_num, neg_samples, samp_bias1, samp_bias2, W1, b1, a1, W2, b2,
           a2, a3, Wd1, bd1, Wd2, bd2):
    x = seq1[0]
    zeros = jnp.zeros((N, D), jnp.float32)

    def edges3(idx, w):
        src = idx[1].astype(jnp.int32).reshape(NW, E // NW)
        dst = idx[0].astype(jnp.int32).reshape(NW, ECPW, 1, EC)
        return src, dst, w.reshape(NW, E // NW)

    src1, dst1, w1e = edges3(adj_index, adj_weight)
    src3, dst3, w3e = edges3(adj_ori_index, adj_ori_weight)

    b1r = b1.reshape(1, D)
    b2r = b2.reshape(1, D)
    a1r = a1.reshape(1, 1)
    a2r = a2.reshape(1, 1)
    a3r = a3.reshape(1, 1)
    bd1r = bd1.reshape(1, 1)
    bd2r = bd2.reshape(1, 1)

    sb1t = samp_bias1.reshape(N, 1)
    sb2t = samp_bias2.T

    h_w, u1 = _proj2(x, W1, Wd1)
    p1 = _spmm_partials(src1, dst1, w1e, h_w, zeros)
    p3 = _spmm_partials(src3, dst3, w3e, h_w, zeros)
    fts2 = _gcn1_tail(p1, b1r, a1r, W2)
    p2 = _spmm_partials(src1, dst1, w1e, fts2, zeros)
    h2, t2, mp_t, lp_t = _stage2(
        p2, b2r, a2r, p3, a3r, Wd2, u1, sb1t, bd1r, bd2r)
    res_mi_pos = mp_t.reshape(1, N)
    res_local_pos = lp_t.reshape(1, N)

    neg_flat = neg_samples.astype(jnp.int32).reshape(-1)
    neg_pad = jnp.concatenate(
        [neg_flat, jnp.zeros((NEG_PAD - NEG * N,), jnp.int32)])
    idx3 = neg_pad.reshape(NW, GCPW, GC)
    g1f, g2f = _gather_rows2(idx3, u1, h2)
    g1 = g1f[:NEG * N].reshape(NEG, N, D)
    g2 = g2f[:NEG * N].reshape(NEG, N, D)

    mn_t, ln_t = _neg_dots(g1, g2, h2, t2, sb2t, bd1r, bd2r)
    res_mi_neg = mn_t.T
    res_local_neg = ln_t.T

    adj_rebuilt = _adj_rebuild(h2)
    return (res_mi_pos, res_mi_neg, res_local_pos, res_local_neg, adj_rebuilt)
